# baseline (speedup 1.0000x reference)
import functools

import jax
import jax.numpy as jnp
from jax import lax
from jax.experimental import pallas as pl
from jax.experimental.pallas import tpu as pltpu
from jax.experimental.pallas import tpu_sc as plsc

_B, _T, _D = 4, 2048, 1024
_NC, _NS = 2, 16
_NW = _NC * _NS
_TPW = _T // _NW
_R = 16
_NSUB = _TPW // _R
_NITEMS = _NSUB * _B
_UNROLL = 4
_CPR = _D // 16


def _sc_body(feat_hbm, sin_hbm, out_hbm, sin_all, fb0, fb1, si0, si1, so0, so1):
    wid = lax.axis_index("s") * _NC + lax.axis_index("c")
    t0 = wid * _TPW

    pltpu.async_copy(feat_hbm.at[0, pl.ds(t0, _R)], fb0, si0)
    pltpu.sync_copy(sin_hbm.at[pl.ds(t0, _TPW)], sin_all)

    def stage(s, i, b, row, cur, si_cur, so_cur, nxt, si_nxt, so_nxt):
        @pl.when(i > 0)
        def _():
            pltpu.make_async_copy(nxt, out_hbm.at[0, pl.ds(0, _R)], so_nxt).wait()

        @pl.when(i < _NITEMS - 1)
        def _():
            last_b = b == _B - 1
            b2 = jnp.where(last_b, 0, b + 1)
            row2 = jnp.where(last_b, row + _R, row)
            pltpu.async_copy(feat_hbm.at[b2, pl.ds(row2, _R)], nxt, si_nxt)

        pltpu.make_async_copy(feat_hbm.at[0, pl.ds(0, _R)], cur, si_cur).wait()

        @plsc.parallel_loop(0, _CPR, 1, unroll=_UNROLL)
        def add_col(c):
            sl = pl.ds(c * 16, 16)
            for r in range(_R):
                plsc.addupdate(cur.at[r, sl], sin_all[s * _R + r, sl])

        pltpu.async_copy(cur, out_hbm.at[b, pl.ds(row, _R)], so_cur)

    for s in range(_NSUB):
        row_s = t0 + s * _R

        def per_b(b, carry, s=s, row_s=row_s):
            i = s * _B + b
            p = lax.bitwise_and(i, 1)

            @pl.when(p == 0)
            def _():
                stage(s, i, b, row_s, fb0, si0, so0, fb1, si1, so1)

            @pl.when(p == 1)
            def _():
                stage(s, i, b, row_s, fb1, si1, so1, fb0, si0, so0)

            return carry

        lax.fori_loop(0, _B, per_b, 0)

    if (_NITEMS - 1) % 2 == 0:
        pltpu.make_async_copy(fb0, out_hbm.at[0, pl.ds(0, _R)], so0).wait()
    else:
        pltpu.make_async_copy(fb1, out_hbm.at[0, pl.ds(0, _R)], so1).wait()


_sc_kernel = functools.partial(
    pl.kernel,
    out_type=jax.ShapeDtypeStruct((_B, _T, _D), jnp.float32),
    mesh=plsc.VectorSubcoreMesh(core_axis_name="c", subcore_axis_name="s"),
    scratch_types=[
        pltpu.VMEM((_TPW, _D), jnp.float32),
        pltpu.VMEM((_R, _D), jnp.float32),
        pltpu.VMEM((_R, _D), jnp.float32),
        pltpu.SemaphoreType.DMA,
        pltpu.SemaphoreType.DMA,
        pltpu.SemaphoreType.DMA,
        pltpu.SemaphoreType.DMA,
    ],
)(_sc_body)


def kernel(features, sinusoids):
    return _sc_kernel(features, sinusoids)


# speedup vs baseline: 1.0089x; 1.0089x over previous
import functools

import jax
import jax.numpy as jnp
from jax import lax
from jax.experimental import pallas as pl
from jax.experimental.pallas import tpu as pltpu
from jax.experimental.pallas import tpu_sc as plsc

_B, _T, _D = 4, 2048, 1024
_NC, _NS = 2, 16
_NW = _NC * _NS
_TPW = _T // _NW
_RPI = 4
_NI = _TPW // _RPI
_BR = _B * _RPI
_UNROLL = 4
_CPR = _D // 16


def _sc_body(feat_hbm, sin_hbm, out_hbm, sin_all, fb0, fb1, si0, si1, so0, so1):
    wid = lax.axis_index("s") * _NC + lax.axis_index("c")
    t0 = wid * _TPW

    def start_in(i, buf, sem):
        for b in range(_B):
            pltpu.async_copy(
                feat_hbm.at[b, pl.ds(t0 + i * _RPI, _RPI)],
                buf.at[pl.ds(b * _RPI, _RPI)], sem)

    def wait_sem(buf, sem):
        for b in range(_B):
            pltpu.make_async_copy(
                feat_hbm.at[0, pl.ds(0, _RPI)],
                buf.at[pl.ds(0, _RPI)], sem).wait()

    def start_out(i, buf, sem):
        for b in range(_B):
            pltpu.async_copy(
                buf.at[pl.ds(b * _RPI, _RPI)],
                out_hbm.at[b, pl.ds(t0 + i * _RPI, _RPI)], sem)

    def wait_out(buf, sem):
        for b in range(_B):
            pltpu.make_async_copy(
                buf.at[pl.ds(0, _RPI)],
                out_hbm.at[0, pl.ds(0, _RPI)], sem).wait()

    start_in(0, fb0, si0)
    pltpu.sync_copy(sin_hbm.at[pl.ds(t0, _TPW)], sin_all)

    for i in range(_NI):
        if i % 2 == 0:
            cur, si_c, so_c, nxt, si_n, so_n = fb0, si0, so0, fb1, si1, so1
        else:
            cur, si_c, so_c, nxt, si_n, so_n = fb1, si1, so1, fb0, si0, so0

        if i > 0:
            wait_out(nxt, so_n)
        if i < _NI - 1:
            start_in(i + 1, nxt, si_n)

        wait_sem(cur, si_c)

        @plsc.parallel_loop(0, _CPR, 1, unroll=_UNROLL)
        def add_col(c, i=i, cur=cur):
            sl = pl.ds(c * 16, 16)
            for r in range(_RPI):
                v = sin_all[i * _RPI + r, sl]
                for b in range(_B):
                    plsc.addupdate(cur.at[b * _RPI + r, sl], v)

        start_out(i, cur, so_c)

    if (_NI - 1) % 2 == 0:
        wait_out(fb0, so0)
    else:
        wait_out(fb1, so1)


_sc_kernel = functools.partial(
    pl.kernel,
    out_type=jax.ShapeDtypeStruct((_B, _T, _D), jnp.float32),
    mesh=plsc.VectorSubcoreMesh(core_axis_name="c", subcore_axis_name="s"),
    scratch_types=[
        pltpu.VMEM((_TPW, _D), jnp.float32),
        pltpu.VMEM((_BR, _D), jnp.float32),
        pltpu.VMEM((_BR, _D), jnp.float32),
        pltpu.SemaphoreType.DMA,
        pltpu.SemaphoreType.DMA,
        pltpu.SemaphoreType.DMA,
        pltpu.SemaphoreType.DMA,
    ],
)(_sc_body)


def kernel(features, sinusoids):
    return _sc_kernel(features, sinusoids)
